# Initial kernel scaffold; baseline (speedup 1.0000x reference)
import jax
import jax.numpy as jnp
from jax.experimental import pallas as pl


def kernel(input, table):
    raise NotImplementedError("write your pallas kernel here")


# speedup vs baseline: 1.8446x; 1.8446x over previous
import functools

import jax
import jax.numpy as jnp
from jax import lax
from jax.experimental import pallas as pl
from jax.experimental.pallas import tpu as pltpu
from jax.experimental.pallas import tpu_sc as plsc

_VOCAB = 1000000
_D = 64
_B = 16384
_H = 50
_NB = _B * _H
_NC = 2
_NS = 16
_NW = _NC * _NS
_PER_W = _NB // _NW
_IDXW = 128
_CH = 1024
_SUB = _CH // _IDXW
_NCH = _PER_W // _CH

_mesh = plsc.VectorSubcoreMesh(core_axis_name="c", subcore_axis_name="s")


@functools.partial(
    pl.kernel,
    mesh=_mesh,
    out_type=jax.ShapeDtypeStruct((_NB, _D), jnp.float32),
    scratch_types=[
        pltpu.VMEM((_SUB, _IDXW), jnp.int32),
        pltpu.VMEM((_CH, _D), jnp.float32),
        pltpu.SemaphoreType.DMA,
    ],
    compiler_params=pltpu.CompilerParams(use_tc_tiling_on_sc=False),
)
def _embed_kernel(idx_hbm, table_hbm, out_hbm, idx_v, rows_v, sem):
    wid = lax.axis_index("s") * _NC + lax.axis_index("c")
    base_row = wid * (_PER_W // _IDXW)

    def chunk_body(ci, _):
        row0 = base_row + ci * _SUB
        pltpu.sync_copy(idx_hbm.at[pl.ds(row0, _SUB)], idx_v)
        copies = []
        for j in range(_SUB):
            copies.append(
                pltpu.async_copy(
                    table_hbm.at[idx_v.at[j]],
                    rows_v.at[pl.ds(j * _IDXW, _IDXW)],
                    sem,
                )
            )
        for cp in copies:
            cp.wait()
        out0 = (base_row + ci * _SUB) * _IDXW
        pltpu.sync_copy(rows_v, out_hbm.at[pl.ds(out0, _CH)])
        return 0

    lax.fori_loop(0, _NCH, chunk_body, 0)


def kernel(input, table):
    idx = input.reshape(_NB // _IDXW, _IDXW).astype(jnp.int32)
    out = _embed_kernel(idx, table)
    return out.reshape(_B, _H, _D)


# speedup vs baseline: 1.8542x; 1.0052x over previous
import functools

import jax
import jax.numpy as jnp
from jax import lax
from jax.experimental import pallas as pl
from jax.experimental.pallas import tpu as pltpu
from jax.experimental.pallas import tpu_sc as plsc

_VOCAB = 1000000
_D = 64
_B = 16384
_H = 50
_NB = _B * _H
_NC = 2
_NS = 16
_NW = _NC * _NS
_PER_W = _NB // _NW
_IDXW = 128
_CH = 512
_SUB = _CH // _IDXW
_NCH = _PER_W // _CH
_NG = _NCH // 2

_mesh = plsc.VectorSubcoreMesh(core_axis_name="c", subcore_axis_name="s")


@functools.partial(
    pl.kernel,
    mesh=_mesh,
    out_type=jax.ShapeDtypeStruct((_NB, _D), jnp.float32),
    scratch_types=[
        pltpu.VMEM((_SUB, _IDXW), jnp.int32),
        pltpu.VMEM((_SUB, _IDXW), jnp.int32),
        pltpu.VMEM((_CH, _D), jnp.float32),
        pltpu.VMEM((_CH, _D), jnp.float32),
        pltpu.SemaphoreType.DMA,
        pltpu.SemaphoreType.DMA,
        pltpu.SemaphoreType.DMA,
        pltpu.SemaphoreType.DMA,
    ],
    compiler_params=pltpu.CompilerParams(use_tc_tiling_on_sc=False),
)
def _embed_kernel(idx_hbm, table_hbm, out_hbm, idx0, idx1, rows0, rows1,
                  gsem0, gsem1, wsem0, wsem1):
    wid = lax.axis_index("s") * _NC + lax.axis_index("c")
    base_row = wid * (_PER_W // _IDXW)
    idx_v = (idx0, idx1)
    rows = (rows0, rows1)
    gsem = (gsem0, gsem1)
    wsem = (wsem0, wsem1)

    def fire_gather(b, ci):
        row0 = base_row + ci * _SUB
        pltpu.sync_copy(idx_hbm.at[pl.ds(row0, _SUB)], idx_v[b])
        for j in range(_SUB):
            pltpu.async_copy(
                table_hbm.at[idx_v[b].at[j]],
                rows[b].at[pl.ds(j * _IDXW, _IDXW)],
                gsem[b],
            )

    def wait_gather(b):
        pltpu.make_async_copy(table_hbm.at[pl.ds(0, _CH)], rows[b], gsem[b]).wait()

    def write_out(b, ci):
        out0 = (base_row + ci * _SUB) * _IDXW
        pltpu.async_copy(rows[b], out_hbm.at[pl.ds(out0, _CH)], wsem[b])

    def wait_write(b):
        pltpu.make_async_copy(rows[b], out_hbm.at[pl.ds(0, _CH)], wsem[b]).wait()

    fire_gather(0, 0)

    def body(g, _):
        c0 = 2 * g

        @pl.when(g > 0)
        def _():
            wait_write(1)

        fire_gather(1, c0 + 1)
        wait_gather(0)
        write_out(0, c0)

        @pl.when(g < _NG - 1)
        def _():
            wait_write(0)
            fire_gather(0, c0 + 2)

        wait_gather(1)
        write_out(1, c0 + 1)
        return 0

    lax.fori_loop(0, _NG, body, 0)
    wait_write(0)
    wait_write(1)


def kernel(input, table):
    idx = input.reshape(_NB // _IDXW, _IDXW).astype(jnp.int32)
    out = _embed_kernel(idx, table)
    return out.reshape(_B, _H, _D)


# speedup vs baseline: 2.9531x; 1.5926x over previous
import functools

import jax
import jax.numpy as jnp
from jax import lax
from jax.experimental import pallas as pl
from jax.experimental.pallas import tpu as pltpu
from jax.experimental.pallas import tpu_sc as plsc

_VOCAB = 1000000
_D = 64
_B = 16384
_H = 50
_NC = 2
_NS = 16
_BT = _B // _NS
_TC = _BT // 128
_DPC = _D // _NC

_mesh = plsc.VectorSubcoreMesh(core_axis_name="c", subcore_axis_name="s")


@functools.partial(
    pl.kernel,
    mesh=_mesh,
    out_type=jax.ShapeDtypeStruct((_H, _D // 8, _B // 128, 8, 128), jnp.float32),
    scratch_types=[
        pltpu.VMEM((_H * _BT,), jnp.int32),
        pltpu.VMEM((_H, 128), jnp.float32),
        pltpu.VMEM_SHARED((_VOCAB,), jnp.float32),
        pltpu.SemaphoreType.DMA,
    ],
)
def _embed_kernel(idx_hbm, table_hbm, out_hbm, idx_res, staging, sprow, sem):
    c = lax.axis_index("c")
    s = lax.axis_index("s")
    b0 = s * _BT

    for h in range(_H):
        pltpu.sync_copy(idx_hbm.at[h, pl.ds(b0, _BT)], idx_res.at[pl.ds(h * _BT, _BT)])

    def plane_body(p, _):
        d = c * _DPC + p
        dr = d // 8
        ds = d % 8

        @pl.when(s == 0)
        def _():
            pltpu.sync_copy(table_hbm.at[d], sprow)

        plsc.subcore_barrier()

        def window_body(j, _):
            def gather_h(h, _):
                pltpu.async_copy(
                    sprow.at[idx_res.at[pl.ds(h * _BT + j * 128, 128)]],
                    staging.at[h],
                    sem,
                )
                return 0

            lax.fori_loop(0, _H, gather_h, 0)
            pltpu.make_async_copy(
                out_hbm.at[:, 0, 0, 0, :], staging, sem
            ).wait()
            pltpu.sync_copy(staging, out_hbm.at[:, dr, s * _TC + j, ds, :])
            return 0

        lax.fori_loop(0, _TC, window_body, 0)
        plsc.subcore_barrier()
        return 0

    lax.fori_loop(0, _DPC, plane_body, 0)


def kernel(input, table):
    out = _embed_kernel(input.T, table.T)
    return out.transpose(2, 4, 0, 1, 3).reshape(_B, _H, _D)


# speedup vs baseline: 3.3734x; 1.1423x over previous
import functools

import jax
import jax.numpy as jnp
from jax import lax
from jax.experimental import pallas as pl
from jax.experimental.pallas import tpu as pltpu
from jax.experimental.pallas import tpu_sc as plsc

_VOCAB = 1000000
_D = 64
_B = 16384
_H = 50
_NC = 2
_NS = 16
_BT = _B // _NS
_TC = _BT // 128
_DPC = _D // _NC

_mesh = plsc.VectorSubcoreMesh(core_axis_name="c", subcore_axis_name="s")


@functools.partial(
    pl.kernel,
    mesh=_mesh,
    out_type=jax.ShapeDtypeStruct((_H, _D // 8, _B // 128, 8, 128), jnp.float32),
    scratch_types=[
        pltpu.VMEM((_H * _BT,), jnp.int32),
        pltpu.VMEM((_H, 128), jnp.float32),
        pltpu.VMEM((_H, 128), jnp.float32),
        pltpu.VMEM_SHARED((_VOCAB,), jnp.float32),
        pltpu.SemaphoreType.DMA,
        pltpu.SemaphoreType.DMA,
        pltpu.SemaphoreType.DMA,
        pltpu.SemaphoreType.DMA,
        pltpu.SemaphoreType.DMA,
    ],
)
def _embed_kernel(idx_hbm, table_hbm, out_hbm, idx_res, st_a, st_b, sprow,
                  gsem_a, gsem_b, wsem_a, wsem_b, ssem):
    c = lax.axis_index("c")
    s = lax.axis_index("s")
    b0 = s * _BT
    st = (st_a, st_b)
    gsem = (gsem_a, gsem_b)
    wsem = (wsem_a, wsem_b)

    for h in range(_H):
        pltpu.sync_copy(idx_hbm.at[h, pl.ds(b0, _BT)], idx_res.at[pl.ds(h * _BT, _BT)])

    def fire(k, j):
        for h in range(_H):
            pltpu.async_copy(
                sprow.at[idx_res.at[pl.ds(h * _BT + j * 128, 128)]],
                st[k].at[h],
                gsem[k],
            )

    def drain(k):
        pltpu.make_async_copy(out_hbm.at[:, 0, 0, 0, :], st[k], gsem[k]).wait()

    def write(k, dr, ds, j):
        pltpu.async_copy(st[k], out_hbm.at[:, dr, s * _TC + j, ds, :], wsem[k])

    def wait_write(k):
        pltpu.make_async_copy(st[k], out_hbm.at[:, 0, 0, 0, :], wsem[k]).wait()

    @pl.when(s == 0)
    def _():
        pltpu.async_copy(table_hbm.at[c * _DPC], sprow, ssem)

    def plane_body(p, _):
        d = c * _DPC + p
        dr = d // 8
        ds = d % 8

        @pl.when(s == 0)
        def _():
            pltpu.make_async_copy(table_hbm.at[0], sprow, ssem).wait()

        plsc.subcore_barrier()
        fire(0, 0)

        def pair_body(g, _):
            @pl.when(g > 0)
            def _():
                wait_write(1)

            fire(1, 2 * g + 1)
            drain(0)
            write(0, dr, ds, 2 * g)

            @pl.when(g < _TC // 2 - 1)
            def _():
                wait_write(0)
                fire(0, 2 * g + 2)

            drain(1)
            write(1, dr, ds, 2 * g + 1)
            return 0

        lax.fori_loop(0, _TC // 2, pair_body, 0)

        plsc.subcore_barrier()

        @pl.when(jnp.logical_and(s == 0, p < _DPC - 1))
        def _():
            pltpu.async_copy(table_hbm.at[d + 1], sprow, ssem)

        wait_write(0)
        wait_write(1)
        return 0

    lax.fori_loop(0, _DPC, plane_body, 0)


def kernel(input, table):
    out = _embed_kernel(input.T, table.T)
    return out.transpose(2, 4, 0, 1, 3).reshape(_B, _H, _D)


# speedup vs baseline: 3.3857x; 1.0037x over previous
import functools

import jax
import jax.numpy as jnp
from jax import lax
from jax.experimental import pallas as pl
from jax.experimental.pallas import tpu as pltpu
from jax.experimental.pallas import tpu_sc as plsc

_VOCAB = 1000000
_D = 64
_B = 16384
_H = 50
_NC = 2
_NS = 16
_BT = _B // _NS
_TC = _BT // 128
_DPC = _D // _NC

_mesh = plsc.VectorSubcoreMesh(core_axis_name="c", subcore_axis_name="s")


@functools.partial(
    pl.kernel,
    mesh=_mesh,
    out_type=jax.ShapeDtypeStruct((_H, _D // 8, _B // 128, 8, 128), jnp.float32),
    scratch_types=[
        pltpu.VMEM((_H * _BT,), jnp.int32),
        pltpu.VMEM((_H, 128), jnp.float32),
        pltpu.VMEM((_H, 128), jnp.float32),
        pltpu.VMEM((128,), jnp.float32),
        pltpu.VMEM_SHARED((_VOCAB,), jnp.float32),
        pltpu.SemaphoreType.DMA,
        pltpu.SemaphoreType.DMA,
        pltpu.SemaphoreType.DMA,
        pltpu.SemaphoreType.DMA,
        pltpu.SemaphoreType.DMA,
    ],
)
def _embed_kernel(idx_hbm, table_hbm, tail_hbm, out_hbm, idx_res, st_a, st_b, tbuf, sprow,
                  gsem_a, gsem_b, wsem_a, wsem_b, ssem):
    c = lax.axis_index("c")
    s = lax.axis_index("s")
    b0 = s * _BT
    st = (st_a, st_b)
    gsem = (gsem_a, gsem_b)
    wsem = (wsem_a, wsem_b)

    for h in range(_H):
        pltpu.sync_copy(idx_hbm.at[h, pl.ds(b0, _BT)], idx_res.at[pl.ds(h * _BT, _BT)])

    def fire(k, j):
        for h in range(_H):
            pltpu.async_copy(
                sprow.at[idx_res.at[pl.ds(h * _BT + j * 128, 128)]],
                st[k].at[h],
                gsem[k],
            )

    def drain(k):
        pltpu.make_async_copy(out_hbm.at[:, 0, 0, 0, :], st[k], gsem[k]).wait()

    def write(k, dr, ds, j):
        pltpu.async_copy(st[k], out_hbm.at[:, dr, s * _TC + j, ds, :], wsem[k])

    def wait_write(k):
        pltpu.make_async_copy(st[k], out_hbm.at[:, 0, 0, 0, :], wsem[k]).wait()

    _CS = 62464
    _CS2 = 62976
    _TOFF = _VOCAB - 128

    def stage_row(d):
        @pl.when(s < 15)
        def _():
            off = s * _CS
            pltpu.async_copy(
                table_hbm.at[d].at[pl.ds(off, _CS)], sprow.at[pl.ds(off, _CS)], ssem
            )

        @pl.when(s == 15)
        def _():
            pltpu.async_copy(
                table_hbm.at[d].at[pl.ds(15 * _CS, _CS2)],
                sprow.at[pl.ds(15 * _CS, _CS2)],
                ssem,
            )
            pltpu.sync_copy(tail_hbm.at[pl.ds(d * 128, 128)], tbuf)
            pltpu.sync_copy(tbuf, sprow.at[pl.ds(_TOFF, 128)])

    def wait_row():
        @pl.when(s < 15)
        def _():
            pltpu.make_async_copy(
                table_hbm.at[0].at[pl.ds(0, _CS)], sprow.at[pl.ds(0, _CS)], ssem
            ).wait()

        @pl.when(s == 15)
        def _():
            pltpu.make_async_copy(
                table_hbm.at[0].at[pl.ds(0, _CS2)], sprow.at[pl.ds(0, _CS2)], ssem
            ).wait()

    stage_row(c * _DPC)

    def plane_body(p, _):
        d = c * _DPC + p
        dr = d // 8
        ds = d % 8

        wait_row()
        plsc.subcore_barrier()
        fire(0, 0)

        def pair_body(g, _):
            @pl.when(g > 0)
            def _():
                wait_write(1)

            fire(1, 2 * g + 1)
            drain(0)
            write(0, dr, ds, 2 * g)

            @pl.when(g < _TC // 2 - 1)
            def _():
                wait_write(0)
                fire(0, 2 * g + 2)

            drain(1)
            write(1, dr, ds, 2 * g + 1)
            return 0

        lax.fori_loop(0, _TC // 2, pair_body, 0)

        plsc.subcore_barrier()

        @pl.when(p < _DPC - 1)
        def _():
            stage_row(d + 1)

        wait_write(0)
        wait_write(1)
        return 0

    lax.fori_loop(0, _DPC, plane_body, 0)


def kernel(input, table):
    tail = table[_VOCAB - 128:].T.reshape(-1)
    out = _embed_kernel(input.T, table.T, tail)
    return out.transpose(2, 4, 0, 1, 3).reshape(_B, _H, _D)


# speedup vs baseline: 3.4995x; 1.0336x over previous
import functools

import jax
import jax.numpy as jnp
from jax import lax
from jax.experimental import pallas as pl
from jax.experimental.pallas import tpu as pltpu
from jax.experimental.pallas import tpu_sc as plsc

_VOCAB = 1000000
_D = 64
_B = 16384
_H = 50
_NC = 2
_NS = 16
_BT = _B // _NS
_TC = _BT // 128
_DPC = _D // _NC

_mesh = plsc.VectorSubcoreMesh(core_axis_name="c", subcore_axis_name="s")


@functools.partial(
    pl.kernel,
    mesh=_mesh,
    out_type=jax.ShapeDtypeStruct((_H, _D // 8, _B // 128, 8, 128), jnp.float32),
    scratch_types=[
        pltpu.VMEM((_H * _BT,), jnp.int32),
        pltpu.VMEM((_H, 128), jnp.float32),
        pltpu.VMEM((_H, 128), jnp.float32),
        pltpu.VMEM((128,), jnp.float32),
        pltpu.VMEM_SHARED((_VOCAB,), jnp.float32),
        pltpu.SemaphoreType.DMA,
        pltpu.SemaphoreType.DMA,
        pltpu.SemaphoreType.DMA,
        pltpu.SemaphoreType.DMA,
        pltpu.SemaphoreType.DMA,
    ],
)
def _embed_kernel(idx_hbm, table_hbm, tail_hbm, out_hbm, idx_res, st_a, st_b, tbuf, sprow,
                  gsem_a, gsem_b, wsem_a, wsem_b, ssem):
    c = lax.axis_index("c")
    s = lax.axis_index("s")
    b0 = s * _BT
    st = (st_a, st_b)
    gsem = (gsem_a, gsem_b)
    wsem = (wsem_a, wsem_b)

    for h in range(_H):
        pltpu.async_copy(
            idx_hbm.at[h, pl.ds(b0, _BT)], idx_res.at[pl.ds(h * _BT, _BT)], gsem_a
        )
    for h in range(_H):
        pltpu.make_async_copy(
            idx_hbm.at[0, pl.ds(0, _BT)], idx_res.at[pl.ds(0, _BT)], gsem_a
        ).wait()

    def fire(k, j):
        for h in range(_H):
            pltpu.async_copy(
                sprow.at[idx_res.at[pl.ds(h * _BT + j * 128, 128)]],
                st[k].at[h],
                gsem[k],
            )

    def drain(k):
        pltpu.make_async_copy(out_hbm.at[:, 0, 0, 0, :], st[k], gsem[k]).wait()

    def write(k, dr, ds, j):
        pltpu.async_copy(st[k], out_hbm.at[:, dr, s * _TC + j, ds, :], wsem[k])

    def wait_write(k):
        pltpu.make_async_copy(st[k], out_hbm.at[:, 0, 0, 0, :], wsem[k]).wait()

    _CS = 62464
    _CS2 = 62976
    _TOFF = _VOCAB - 128

    def stage_row(d):
        @pl.when(s < 15)
        def _():
            off = s * _CS
            pltpu.async_copy(
                table_hbm.at[d].at[pl.ds(off, _CS)], sprow.at[pl.ds(off, _CS)], ssem
            )

        @pl.when(s == 15)
        def _():
            pltpu.async_copy(
                table_hbm.at[d].at[pl.ds(15 * _CS, _CS2)],
                sprow.at[pl.ds(15 * _CS, _CS2)],
                ssem,
            )
            pltpu.sync_copy(tail_hbm.at[pl.ds(d * 128, 128)], tbuf)
            pltpu.sync_copy(tbuf, sprow.at[pl.ds(_TOFF, 128)])

    def wait_row():
        @pl.when(s < 15)
        def _():
            pltpu.make_async_copy(
                table_hbm.at[0].at[pl.ds(0, _CS)], sprow.at[pl.ds(0, _CS)], ssem
            ).wait()

        @pl.when(s == 15)
        def _():
            pltpu.make_async_copy(
                table_hbm.at[0].at[pl.ds(0, _CS2)], sprow.at[pl.ds(0, _CS2)], ssem
            ).wait()

    stage_row(c * _DPC)

    def plane_body(p, _):
        d = c * _DPC + p
        dr = d // 8
        ds = d % 8

        wait_row()
        plsc.subcore_barrier()
        fire(0, 0)

        def pair_body(g, _):
            @pl.when(g > 0)
            def _():
                wait_write(1)

            fire(1, 2 * g + 1)
            drain(0)
            write(0, dr, ds, 2 * g)

            @pl.when(g < _TC // 2 - 1)
            def _():
                wait_write(0)
                fire(0, 2 * g + 2)

            drain(1)
            write(1, dr, ds, 2 * g + 1)
            return 0

        lax.fori_loop(0, _TC // 2, pair_body, 0)

        plsc.subcore_barrier()

        @pl.when(p < _DPC - 1)
        def _():
            stage_row(d + 1)

        wait_write(0)
        wait_write(1)
        return 0

    lax.fori_loop(0, _DPC, plane_body, 0)


def kernel(input, table):
    tail = table[_VOCAB - 128:].T.reshape(-1)
    out = _embed_kernel(input.T, table.T, tail)
    return out.transpose(2, 4, 0, 1, 3).reshape(_B, _H, _D)
